# baseline (speedup 1.0000x reference)
import functools

import jax
import jax.numpy as jnp
from jax import lax
from jax.experimental import pallas as pl
from jax.experimental.pallas import tpu as pltpu
from jax.experimental.pallas import tpu_sc as plsc

F32 = jnp.float32

NC = 2
NS = 16
CH = 128


def _relu(v):
    return jnp.maximum(v, 0.0)


def _tc_node_features(xp, posp, T0, npad, nf, n_off):
    B = 1024
    grid = npad // B

    def body(x_ref, pos_ref, t_ref, o_ref):
        xb = x_ref[...]
        iota = lax.broadcasted_iota(jnp.int32, (1, nf), 1)
        acc = jnp.zeros((B, nf), F32)
        for i in range(9):
            gi = xb[:, i:i + 1].astype(jnp.int32) + n_off[i]
            acc = acc + (gi == iota).astype(F32)
        acc = acc + xb[:, 9:10] * (iota == n_off[9]).astype(F32)
        pb = pos_ref[...]
        for j in range(3):
            acc = acc + pb[:, j:j + 1] * (iota == (n_off[9] + 1 + j)).astype(F32)
        o_ref[...] = jnp.dot(acc, t_ref[...], preferred_element_type=F32)

    return pl.pallas_call(
        body,
        grid=(grid,),
        in_specs=[
            pl.BlockSpec((B, 13), lambda i: (i, 0)),
            pl.BlockSpec((B, 3), lambda i: (i, 0)),
            pl.BlockSpec((nf, 128), lambda i: (0, 0)),
        ],
        out_specs=pl.BlockSpec((B, 128), lambda i: (i, 0)),
        out_shape=jax.ShapeDtypeStruct((npad, 128), F32),
        compiler_params=pltpu.CompilerParams(dimension_semantics=("arbitrary",)),
    )(xp, posp, T0)


NB = 8
CW = 64


_SC_CACHE = {}


def _sc_scatter_add(h, src2, dst4, npad, epad):
    acch = npad // 4 + NS
    zeros = jnp.zeros((acch // NS, 128), F32)
    key = ("scatter", npad, epad)
    if key not in _SC_CACHE:
        _SC_CACHE[key] = _build_sc_scatter(npad, epad)
    return _SC_CACHE[key](zeros, h, src2, dst4)


def _build_sc_scatter(npad, epad):
    nrows = epad // CW
    nch = nrows // NS
    Q = npad // 4
    acch = Q + NS
    rpz = acch // NS
    rps = Q // NS
    mesh = plsc.VectorSubcoreMesh(core_axis_name="c", subcore_axis_name="s")

    @functools.partial(
        pl.kernel,
        out_type=jax.ShapeDtypeStruct((npad, 128), F32),
        mesh=mesh,
        scratch_types=[
            pltpu.VMEM((nch, CW), jnp.int32),
            pltpu.VMEM((nch, CW), jnp.int32),
            [pltpu.VMEM((CW, 128), F32)] * NB,
            pltpu.VMEM_SHARED((acch, 128), F32),
            [pltpu.SemaphoreType.DMA] * NB,
            [pltpu.SemaphoreType.DMA] * NB,
        ],
    )
    def k(zeros_hbm, h_hbm, src_hbm, dst_hbm, out_hbm, sidx, didx, rows,
          acc, gsem, ssem):
        c = lax.axis_index("c")
        s = lax.axis_index("s")
        crow = s * nch
        pltpu.sync_copy(src_hbm.at[pl.ds(crow, nch)], sidx)

        for p in range(2):
            q = p * NC + c
            pltpu.sync_copy(dst_hbm.at[pl.ds(q * nrows + crow, nch)], didx)
            pltpu.sync_copy(zeros_hbm, acc.at[pl.ds(s * rpz, rpz)])
            plsc.subcore_barrier()

            for b in range(NB):
                pltpu.async_copy(h_hbm.at[sidx.at[b]], rows[b], gsem[b])

            def step(i, carry):
                for b in range(NB):
                    j = i * NB + b
                    pltpu.make_async_copy(h_hbm.at[sidx.at[j]], rows[b],
                                          gsem[b]).wait()
                    pltpu.async_copy(rows[b], acc.at[didx.at[j]], ssem[b],
                                     add=True)

                    @pl.when(j + NB < nch)
                    def _():
                        pltpu.make_async_copy(rows[b], acc.at[didx.at[j]],
                                              ssem[b]).wait()
                        pltpu.async_copy(h_hbm.at[sidx.at[j + NB]], rows[b],
                                         gsem[b])
                return carry

            lax.fori_loop(0, nch // NB, step, 0)
            for b in range(NB):
                pltpu.make_async_copy(rows[b], acc.at[didx.at[nch - NB + b]],
                                      ssem[b]).wait()
            plsc.subcore_barrier()
            pltpu.sync_copy(acc.at[pl.ds(s * rps, rps)],
                            out_hbm.at[pl.ds(q * Q + s * rps, rps)])
            plsc.subcore_barrier()

    return k


def _sc_gather2(u, src2, dst2, epad):
    nch = epad // CW // (NC * NS)
    mesh = plsc.VectorSubcoreMesh(core_axis_name="c", subcore_axis_name="s")
    sds = jax.ShapeDtypeStruct((epad, 128), F32)
    NP = NB // 2

    @functools.partial(
        pl.kernel,
        out_type=(sds, sds),
        mesh=mesh,
        scratch_types=[
            pltpu.VMEM((nch, CW), jnp.int32),
            pltpu.VMEM((nch, CW), jnp.int32),
            [pltpu.VMEM((CW, 128), F32)] * NB,
            [pltpu.SemaphoreType.DMA] * NB,
            [pltpu.SemaphoreType.DMA] * NB,
        ],
    )
    def k(u_hbm, src_hbm, dst_hbm, gs_hbm, gd_hbm, sidx, didx, rows,
          gsem, wsem):
        c = lax.axis_index("c")
        s = lax.axis_index("s")
        wid = c * NS + s
        crow = wid * nch
        base = crow * CW
        pltpu.sync_copy(src_hbm.at[pl.ds(crow, nch)], sidx)
        pltpu.sync_copy(dst_hbm.at[pl.ds(crow, nch)], didx)

        idxs = [sidx] * NP + [didx] * NP

        def fire(b, j):
            pltpu.async_copy(u_hbm.at[idxs[b].at[j]], rows[b], gsem[b])

        for b in range(NB):
            fire(b, b % NP)

        def step(i, carry):
            for b in range(NB):
                p = b % NP
                j = i * NP + p
                out = gs_hbm if b < NP else gd_hbm
                pltpu.make_async_copy(u_hbm.at[idxs[b].at[j]], rows[b],
                                      gsem[b]).wait()
                pltpu.async_copy(rows[b], out.at[pl.ds(base + j * CW, CW)],
                                 wsem[b])

                @pl.when(j + NP < nch)
                def _():
                    pltpu.make_async_copy(
                        rows[b], out.at[pl.ds(base + j * CW, CW)],
                        wsem[b]).wait()
                    fire(b, j + NP)
            return carry

        lax.fori_loop(0, nch // NP, step, 0)
        for b in range(NB):
            p = b % NP
            out = gs_hbm if b < NP else gd_hbm
            pltpu.make_async_copy(
                rows[b], out.at[pl.ds(base + (nch - NP + p) * CW, CW)],
                wsem[b]).wait()

    return k(u, src2, dst2)


def _tc_gin(hin, agg, W1, b1, W2f, b2f, ST, npad, n, fold_bn):
    B = 1024
    grid = npad // B

    def body(h_ref, ag_ref, w1_ref, b1_ref, w2_ref, b2_ref, st_ref,
             u_ref, stat_ref):
        pid = pl.program_id(0)
        h = h_ref[...]
        ag = ag_ref[...]
        if fold_bn:
            S = st_ref[0:1, :]
            T = st_ref[1:2, :]
            onehot108 = (lax.broadcasted_iota(jnp.int32, (1, 128), 1) == 108)
            indeg = jnp.sum(jnp.where(onehot108, ag, 0.0), axis=1, keepdims=True)
            z = h * S + T + S * ag + T * indeg
        else:
            z = h + ag
        t = _relu(jnp.dot(z, w1_ref[...], preferred_element_type=F32)
                  + b1_ref[0:1, :])
        u = _relu(jnp.dot(t, w2_ref[...], preferred_element_type=F32)
                  + b2_ref[0:1, :])
        u_ref[...] = u
        rid = lax.broadcasted_iota(jnp.int32, (B, 1), 0) + pid * B
        um = jnp.where(rid < n, u, 0.0)
        ssum = jnp.sum(um, axis=0, keepdims=True)
        sq = jnp.sum(um * um, axis=0, keepdims=True)

        @pl.when(pid == 0)
        def _():
            stat_ref[...] = jnp.zeros((8, 128), F32)

        upd = jnp.concatenate([ssum, sq, jnp.zeros((6, 128), F32)], axis=0)
        stat_ref[...] = stat_ref[...] + upd

    return pl.pallas_call(
        body,
        grid=(grid,),
        in_specs=[
            pl.BlockSpec((B, 128), lambda i: (i, 0)),
            pl.BlockSpec((B, 128), lambda i: (i, 0)),
            pl.BlockSpec((128, 108), lambda i: (0, 0)),
            pl.BlockSpec((8, 108), lambda i: (0, 0)),
            pl.BlockSpec((108, 128), lambda i: (0, 0)),
            pl.BlockSpec((8, 128), lambda i: (0, 0)),
            pl.BlockSpec((8, 128), lambda i: (0, 0)),
        ],
        out_specs=[
            pl.BlockSpec((B, 128), lambda i: (i, 0)),
            pl.BlockSpec((8, 128), lambda i: (0, 0)),
        ],
        out_shape=[
            jax.ShapeDtypeStruct((npad, 128), F32),
            jax.ShapeDtypeStruct((8, 128), F32),
        ],
        compiler_params=pltpu.CompilerParams(dimension_semantics=("arbitrary",)),
    )(hin, agg, W1, b1, W2f, b2f, ST)


def _tc_head(gs, gd, eap, ebp, ST2, M1, be1, We2, be2, We3, be3,
             Wh1a, Wh1b, Wh1c, bh1, Wh2, bh2, Wh3, bh3, wh4, misc,
             epad, e_count, ef, e_off, g):
    B = 512
    grid = epad // B

    def body(gs_ref, gd_ref, ea_ref, eb_ref, st_ref, m1_ref, be1_ref,
             we2_ref, be2_ref, we3_ref, be3_ref, wa_ref, wb_ref, wc_ref,
             bh1_ref, wh2_ref, bh2_ref, wh3_ref, bh3_ref, wh4_ref, misc_ref,
             out_ref):
        pid = pl.program_id(0)
        ea = ea_ref[...]
        iote = lax.broadcasted_iota(jnp.int32, (1, ef), 1)
        acc = jnp.zeros((B, ef), F32)
        for i in range(5):
            gi = ea[:, i:i + 1].astype(jnp.int32) + e_off[i]
            acc = acc + (gi == iote).astype(F32)
        acc = acc + ea[:, 5:6] * (iote == e_off[5]).astype(F32)
        e1 = _relu(jnp.dot(acc, m1_ref[...], preferred_element_type=F32)
                   + be1_ref[0:1, :])
        e2 = _relu(jnp.dot(e1, we2_ref[...], preferred_element_type=F32)
                   + be2_ref[0:1, :])
        e3 = jnp.dot(e2, we3_ref[...], preferred_element_type=F32) + be3_ref[0:1, :]
        S = st_ref[0:1, :]
        T = st_ref[1:2, :]
        xs = gs_ref[...] * S + T
        xd = gd_ref[...] * S + T
        z1 = _relu(jnp.dot(xs, wa_ref[...], preferred_element_type=F32)
                   + jnp.dot(xd, wb_ref[...], preferred_element_type=F32)
                   + jnp.dot(e3, wc_ref[...], preferred_element_type=F32)
                   + bh1_ref[0:1, :])
        z2 = _relu(jnp.dot(z1, wh2_ref[...], preferred_element_type=F32)
                   + bh2_ref[0:1, :])
        z3 = _relu(jnp.dot(z2, wh3_ref[...], preferred_element_type=F32)
                   + bh3_ref[0:1, :])
        z4 = jnp.sum(z3 * wh4_ref[0:1, :], axis=1, keepdims=True) \
            + misc_ref[0:1, 0:1]
        rid = lax.broadcasted_iota(jnp.int32, (B, 1), 0) + pid * B
        z4 = jnp.where(rid < e_count, z4, 0.0)
        iog = lax.broadcasted_iota(jnp.int32, (1, g), 1).astype(F32)
        onehot = (eb_ref[...] == iog).astype(F32)
        pooled = lax.dot_general(z4, onehot, (((0,), (0,)), ((), ())),
                                 preferred_element_type=F32)

        @pl.when(pid == 0)
        def _():
            out_ref[...] = jnp.zeros((8, g), F32)

        out_ref[...] = out_ref[...] + jnp.concatenate(
            [pooled, jnp.zeros((7, g), F32)], axis=0)

    full = lambda i: (0, 0)
    return pl.pallas_call(
        body,
        grid=(grid,),
        in_specs=[
            pl.BlockSpec((B, 128), lambda i: (i, 0)),
            pl.BlockSpec((B, 128), lambda i: (i, 0)),
            pl.BlockSpec((B, 6), lambda i: (i, 0)),
            pl.BlockSpec((B, 1), lambda i: (i, 0)),
            pl.BlockSpec((8, 128), full),
            pl.BlockSpec((ef, 22), full),
            pl.BlockSpec((8, 22), full),
            pl.BlockSpec((22, 40), full),
            pl.BlockSpec((8, 40), full),
            pl.BlockSpec((40, 40), full),
            pl.BlockSpec((8, 40), full),
            pl.BlockSpec((128, 512), full),
            pl.BlockSpec((128, 512), full),
            pl.BlockSpec((40, 512), full),
            pl.BlockSpec((8, 512), full),
            pl.BlockSpec((512, 512), full),
            pl.BlockSpec((8, 512), full),
            pl.BlockSpec((512, 256), full),
            pl.BlockSpec((8, 256), full),
            pl.BlockSpec((8, 256), full),
            pl.BlockSpec((8, 8), full),
        ],
        out_specs=pl.BlockSpec((8, g), full),
        out_shape=jax.ShapeDtypeStruct((8, g), F32),
        compiler_params=pltpu.CompilerParams(dimension_semantics=("arbitrary",)),
    )(gs, gd, eap, ebp, ST2, M1, be1, We2, be2, We3, be3,
      Wh1a, Wh1b, Wh1c, bh1, Wh2, bh2, Wh3, bh3, wh4, misc)


def _pad_bias(b, n):
    out = jnp.zeros((8, n), F32)
    return out.at[0, :b.shape[0]].set(b)


def kernel(x, edge_index, edge_attr, edge_batch, pos, params):
    n = x.shape[0]
    e = edge_index.shape[1]
    g = 64
    npad = -(-n // 1024) * 1024
    epad = -(-e // (NC * NS * CH)) * (NC * NS * CH)

    node_tabs = params['node_emb']
    edge_tabs = params['edge_emb']
    nv = [t.shape[0] for t in node_tabs]
    nd = [t.shape[1] for t in node_tabs]
    ev = [t.shape[0] for t in edge_tabs]
    ed = [t.shape[1] for t in edge_tabs]
    n_voff = [0]
    for v in nv:
        n_voff.append(n_voff[-1] + v)
    n_doff = [0]
    for dd in nd:
        n_doff.append(n_doff[-1] + dd)
    e_voff = [0]
    for v in ev:
        e_voff.append(e_voff[-1] + v)
    e_doff = [0]
    for dd in ed:
        e_doff.append(e_doff[-1] + dd)
    nf = -(-(n_voff[-1] + 4) // 8) * 8
    ef = -(-(e_voff[-1] + 1) // 8) * 8

    T0 = jnp.zeros((nf, 128), F32)
    for i in range(9):
        T0 = T0.at[n_voff[i]:n_voff[i] + nv[i],
                   n_doff[i]:n_doff[i] + nd[i]].set(node_tabs[i])
    T0 = T0.at[n_voff[-1], n_doff[-1]].set(1.0)
    for j in range(3):
        T0 = T0.at[n_voff[-1] + 1 + j, n_doff[-1] + 1 + j].set(0.1)
    node_onehot_off = n_voff[:9] + [n_voff[-1]]
    edge_onehot_off = e_voff[:5] + [e_voff[-1]]

    ea_map = jnp.zeros((ef, 11), F32)
    for i in range(5):
        ea_map = ea_map.at[e_voff[i]:e_voff[i] + ev[i],
                           e_doff[i]:e_doff[i] + ed[i]].set(edge_tabs[i])
    ea_map = ea_map.at[e_voff[-1], 10].set(0.1)
    M1 = ea_map @ params['e_lin'][0]['w'].T

    W1 = jnp.zeros((128, 108), F32).at[:37].set(params['x_nn1'][0]['w'].T)
    b1 = _pad_bias(params['x_nn1'][0]['b'], 108)
    W2f = jnp.zeros((108, 128), F32).at[:, :108].set(params['x_nn1'][1]['w'].T)
    b2f = _pad_bias(params['x_nn1'][1]['b'], 128).at[0, 108].set(1.0)
    W3 = jnp.zeros((128, 108), F32).at[:108].set(params['x_nn2'][0]['w'].T)
    b3 = _pad_bias(params['x_nn2'][0]['b'], 108)
    W4f = jnp.zeros((108, 128), F32).at[:, :108].set(params['x_nn2'][1]['w'].T)
    b4f = _pad_bias(params['x_nn2'][1]['b'], 128)

    h0w = params['head'][0]['w']
    Wh1a = jnp.zeros((128, 512), F32).at[:108].set(h0w[:, :108].T)
    Wh1b = jnp.zeros((128, 512), F32).at[:108].set(h0w[:, 108:216].T)
    Wh1c = h0w[:, 216:256].T
    bh1 = _pad_bias(params['head'][0]['b'], 512)
    Wh2 = params['head'][1]['w'].T
    bh2 = _pad_bias(params['head'][1]['b'], 512)
    Wh3 = params['head'][2]['w'].T
    bh3 = _pad_bias(params['head'][2]['b'], 256)
    wh4 = jnp.zeros((8, 256), F32).at[0].set(params['head'][3]['w'][0])
    misc = jnp.zeros((8, 8), F32).at[0, 0].set(params['head'][3]['b'][0])
    be1 = _pad_bias(params['e_lin'][0]['b'], 22)
    We2 = params['e_lin'][1]['w'].T
    be2 = _pad_bias(params['e_lin'][1]['b'], 40)
    We3 = params['e_lin'][2]['w'].T
    be3 = _pad_bias(params['e_lin'][2]['b'], 40)

    xp = jnp.zeros((npad, 13), F32).at[:n].set(x)
    posp = jnp.zeros((npad, 3), F32).at[:n].set(pos)
    srcp = jnp.full((epad,), n, jnp.int32).at[:e].set(edge_index[0])
    dstp = jnp.full((epad,), n, jnp.int32).at[:e].set(edge_index[1])
    src2 = srcp.reshape(-1, CW)
    dst2 = dstp.reshape(-1, CW)
    qq = npad // 4
    dst4 = jnp.concatenate(
        [jnp.where(dstp // qq == q, dstp - q * qq, qq).reshape(-1, CW)
         for q in range(4)], axis=0)
    eap = jnp.zeros((epad, 6), F32).at[:e].set(edge_attr)
    ebp = jnp.zeros((epad, 1), F32).at[:e, 0].set(edge_batch.astype(F32))

    h0 = _tc_node_features(xp, posp, T0, npad, nf, node_onehot_off)
    agg0 = _sc_scatter_add(h0, src2, dst4, npad, epad)
    dummy_st = jnp.zeros((8, 128), F32)
    u1, st1 = _tc_gin(h0, agg0, W1, b1, W2f, b2f,
                      dummy_st, npad, n, fold_bn=False)
    mean1 = st1[0, :108] / n
    var1 = st1[1, :108] / n - mean1 * mean1
    s1 = params['bn1']['g'] / jnp.sqrt(var1 + 1e-5)
    t1 = params['bn1']['b'] - mean1 * s1
    ST1 = jnp.zeros((8, 128), F32).at[0, :108].set(s1).at[1, :108].set(t1)

    aggu = _sc_scatter_add(u1, src2, dst4, npad, epad)
    u2, st2 = _tc_gin(u1, aggu, W3, b3, W4f, b4f,
                      ST1, npad, n, fold_bn=True)
    mean2 = st2[0, :108] / n
    var2 = st2[1, :108] / n - mean2 * mean2
    s2 = params['bn2']['g'] / jnp.sqrt(var2 + 1e-5)
    t2 = params['bn2']['b'] - mean2 * s2
    ST2 = jnp.zeros((8, 128), F32).at[0, :108].set(s2).at[1, :108].set(t2)

    gs, gd = _sc_gather2(u2, src2, dst2, epad)
    pooled = _tc_head(gs, gd, eap, ebp, ST2, M1, be1, We2, be2, We3, be3,
                      Wh1a, Wh1b, Wh1c, bh1, Wh2, bh2, Wh3, bh3, wh4, misc,
                      epad, e, ef, edge_onehot_off, g)
    return pooled[0, :].reshape(g, 1)


# speedup vs baseline: 1.5044x; 1.5044x over previous
import functools

import jax
import jax.numpy as jnp
from jax import lax
from jax.experimental import pallas as pl
from jax.experimental.pallas import tpu as pltpu
from jax.experimental.pallas import tpu_sc as plsc

F32 = jnp.float32

NC = 2
NS = 16
CH = 128


def _relu(v):
    return jnp.maximum(v, 0.0)


def _tc_node_features(xp, posp, T0, npad, nf, n_off):
    B = 1024
    grid = npad // B

    def body(x_ref, pos_ref, t_ref, o_ref):
        xb = x_ref[...]
        iota = lax.broadcasted_iota(jnp.int32, (1, nf), 1)
        acc = jnp.zeros((B, nf), F32)
        for i in range(9):
            gi = xb[:, i:i + 1].astype(jnp.int32) + n_off[i]
            acc = acc + (gi == iota).astype(F32)
        acc = acc + xb[:, 9:10] * (iota == n_off[9]).astype(F32)
        pb = pos_ref[...]
        for j in range(3):
            acc = acc + pb[:, j:j + 1] * (iota == (n_off[9] + 1 + j)).astype(F32)
        o_ref[...] = jnp.dot(acc, t_ref[...], preferred_element_type=F32)

    return pl.pallas_call(
        body,
        grid=(grid,),
        in_specs=[
            pl.BlockSpec((B, 13), lambda i: (i, 0)),
            pl.BlockSpec((B, 3), lambda i: (i, 0)),
            pl.BlockSpec((nf, 128), lambda i: (0, 0)),
        ],
        out_specs=pl.BlockSpec((B, 128), lambda i: (i, 0)),
        out_shape=jax.ShapeDtypeStruct((npad, 128), F32),
        compiler_params=pltpu.CompilerParams(dimension_semantics=("arbitrary",)),
    )(xp, posp, T0)


NB = 8
CW = 64


_SC_CACHE = {}


CWS = 128


def _sc_scatter_add(h, src2s, dst2s, npad, epad):
    zeros = jnp.zeros((npad // NS, 128), F32)
    key = ("scatter", npad, epad)
    if key not in _SC_CACHE:
        _SC_CACHE[key] = _build_sc_scatter(npad, epad)
    return _SC_CACHE[key](zeros, h, src2s, dst2s)


def _build_sc_scatter(npad, epad):
    nrows = epad // CWS
    nch = nrows // (NC * NS)
    rps = npad // NS
    mesh = plsc.VectorSubcoreMesh(core_axis_name="c", subcore_axis_name="s")

    @functools.partial(
        pl.kernel,
        out_type=jax.ShapeDtypeStruct((NC * npad, 128), F32),
        mesh=mesh,
        scratch_types=[
            pltpu.VMEM((nch, CWS), jnp.int32),
            pltpu.VMEM((nch, CWS), jnp.int32),
            pltpu.VMEM((CWS, 128), F32),
            pltpu.VMEM_SHARED((npad, 128), F32),
            pltpu.SemaphoreType.DMA,
        ],
    )
    def k(zeros_hbm, h_hbm, src_hbm, dst_hbm, out_hbm, sidx, didx, rows,
          acc, sem):
        c = lax.axis_index("c")
        s = lax.axis_index("s")
        wid = c * NS + s
        crow = wid * nch
        r0 = s * rps
        pltpu.sync_copy(src_hbm.at[pl.ds(crow, nch)], sidx)
        pltpu.sync_copy(dst_hbm.at[pl.ds(crow, nch)], didx)
        pltpu.sync_copy(zeros_hbm, acc.at[pl.ds(r0, rps)])
        plsc.subcore_barrier()

        def step(j, carry):
            pltpu.async_copy(h_hbm.at[sidx.at[j]], rows, sem).wait()
            pltpu.sync_copy(rows, acc.at[didx.at[j]], add=True)
            return carry

        lax.fori_loop(0, nch, step, 0)
        plsc.subcore_barrier()
        pltpu.sync_copy(acc.at[pl.ds(r0, rps)],
                        out_hbm.at[pl.ds(c * npad + r0, rps)])

    return k


def _sc_gather2(u, src2, dst2, epad):
    nch = epad // CW // (NC * NS)
    mesh = plsc.VectorSubcoreMesh(core_axis_name="c", subcore_axis_name="s")
    sds = jax.ShapeDtypeStruct((epad, 128), F32)
    NP = NB // 2

    @functools.partial(
        pl.kernel,
        out_type=(sds, sds),
        mesh=mesh,
        scratch_types=[
            pltpu.VMEM((nch, CW), jnp.int32),
            pltpu.VMEM((nch, CW), jnp.int32),
            [pltpu.VMEM((CW, 128), F32)] * NB,
            [pltpu.SemaphoreType.DMA] * NB,
            [pltpu.SemaphoreType.DMA] * NB,
        ],
    )
    def k(u_hbm, src_hbm, dst_hbm, gs_hbm, gd_hbm, sidx, didx, rows,
          gsem, wsem):
        c = lax.axis_index("c")
        s = lax.axis_index("s")
        wid = c * NS + s
        crow = wid * nch
        base = crow * CW
        pltpu.sync_copy(src_hbm.at[pl.ds(crow, nch)], sidx)
        pltpu.sync_copy(dst_hbm.at[pl.ds(crow, nch)], didx)

        idxs = [sidx] * NP + [didx] * NP

        def fire(b, j):
            pltpu.async_copy(u_hbm.at[idxs[b].at[j]], rows[b], gsem[b])

        for b in range(NB):
            fire(b, b % NP)

        def step(i, carry):
            for b in range(NB):
                p = b % NP
                j = i * NP + p
                out = gs_hbm if b < NP else gd_hbm
                pltpu.make_async_copy(u_hbm.at[idxs[b].at[j]], rows[b],
                                      gsem[b]).wait()
                pltpu.async_copy(rows[b], out.at[pl.ds(base + j * CW, CW)],
                                 wsem[b])

                @pl.when(j + NP < nch)
                def _():
                    pltpu.make_async_copy(
                        rows[b], out.at[pl.ds(base + j * CW, CW)],
                        wsem[b]).wait()
                    fire(b, j + NP)
            return carry

        lax.fori_loop(0, nch // NP, step, 0)
        for b in range(NB):
            p = b % NP
            out = gs_hbm if b < NP else gd_hbm
            pltpu.make_async_copy(
                rows[b], out.at[pl.ds(base + (nch - NP + p) * CW, CW)],
                wsem[b]).wait()

    return k(u, src2, dst2)


def _tc_gin(hin, agg_a, agg_b, W1, b1, W2f, b2f, ST, npad, n, fold_bn):
    B = 1024
    grid = npad // B

    def body(h_ref, aa_ref, ab_ref, w1_ref, b1_ref, w2_ref, b2_ref, st_ref,
             u_ref, stat_ref):
        pid = pl.program_id(0)
        h = h_ref[...]
        ag = aa_ref[...] + ab_ref[...]
        if fold_bn:
            S = st_ref[0:1, :]
            T = st_ref[1:2, :]
            onehot108 = (lax.broadcasted_iota(jnp.int32, (1, 128), 1) == 108)
            indeg = jnp.sum(jnp.where(onehot108, ag, 0.0), axis=1, keepdims=True)
            z = h * S + T + S * ag + T * indeg
        else:
            z = h + ag
        t = _relu(jnp.dot(z, w1_ref[...], preferred_element_type=F32)
                  + b1_ref[0:1, :])
        u = _relu(jnp.dot(t, w2_ref[...], preferred_element_type=F32)
                  + b2_ref[0:1, :])
        u_ref[...] = u
        rid = lax.broadcasted_iota(jnp.int32, (B, 1), 0) + pid * B
        um = jnp.where(rid < n, u, 0.0)
        ssum = jnp.sum(um, axis=0, keepdims=True)
        sq = jnp.sum(um * um, axis=0, keepdims=True)

        @pl.when(pid == 0)
        def _():
            stat_ref[...] = jnp.zeros((8, 128), F32)

        upd = jnp.concatenate([ssum, sq, jnp.zeros((6, 128), F32)], axis=0)
        stat_ref[...] = stat_ref[...] + upd

    return pl.pallas_call(
        body,
        grid=(grid,),
        in_specs=[
            pl.BlockSpec((B, 128), lambda i: (i, 0)),
            pl.BlockSpec((B, 128), lambda i: (i, 0)),
            pl.BlockSpec((B, 128), lambda i: (i, 0)),
            pl.BlockSpec((128, 108), lambda i: (0, 0)),
            pl.BlockSpec((8, 108), lambda i: (0, 0)),
            pl.BlockSpec((108, 128), lambda i: (0, 0)),
            pl.BlockSpec((8, 128), lambda i: (0, 0)),
            pl.BlockSpec((8, 128), lambda i: (0, 0)),
        ],
        out_specs=[
            pl.BlockSpec((B, 128), lambda i: (i, 0)),
            pl.BlockSpec((8, 128), lambda i: (0, 0)),
        ],
        out_shape=[
            jax.ShapeDtypeStruct((npad, 128), F32),
            jax.ShapeDtypeStruct((8, 128), F32),
        ],
        compiler_params=pltpu.CompilerParams(dimension_semantics=("arbitrary",)),
    )(hin, agg_a, agg_b, W1, b1, W2f, b2f, ST)


def _tc_head(gs, gd, eap, ebp, ST2, M1, be1, We2, be2, We3, be3,
             Wh1a, Wh1b, Wh1c, bh1, Wh2, bh2, Wh3, bh3, wh4, misc,
             epad, e_count, ef, e_off, g):
    B = 512
    grid = epad // B

    def body(gs_ref, gd_ref, ea_ref, eb_ref, st_ref, m1_ref, be1_ref,
             we2_ref, be2_ref, we3_ref, be3_ref, wa_ref, wb_ref, wc_ref,
             bh1_ref, wh2_ref, bh2_ref, wh3_ref, bh3_ref, wh4_ref, misc_ref,
             out_ref):
        pid = pl.program_id(0)
        ea = ea_ref[...]
        iote = lax.broadcasted_iota(jnp.int32, (1, ef), 1)
        acc = jnp.zeros((B, ef), F32)
        for i in range(5):
            gi = ea[:, i:i + 1].astype(jnp.int32) + e_off[i]
            acc = acc + (gi == iote).astype(F32)
        acc = acc + ea[:, 5:6] * (iote == e_off[5]).astype(F32)
        e1 = _relu(jnp.dot(acc, m1_ref[...], preferred_element_type=F32)
                   + be1_ref[0:1, :])
        e2 = _relu(jnp.dot(e1, we2_ref[...], preferred_element_type=F32)
                   + be2_ref[0:1, :])
        e3 = jnp.dot(e2, we3_ref[...], preferred_element_type=F32) + be3_ref[0:1, :]
        S = st_ref[0:1, :]
        T = st_ref[1:2, :]
        xs = gs_ref[...] * S + T
        xd = gd_ref[...] * S + T
        z1 = _relu(jnp.dot(xs, wa_ref[...], preferred_element_type=F32)
                   + jnp.dot(xd, wb_ref[...], preferred_element_type=F32)
                   + jnp.dot(e3, wc_ref[...], preferred_element_type=F32)
                   + bh1_ref[0:1, :])
        z2 = _relu(jnp.dot(z1, wh2_ref[...], preferred_element_type=F32)
                   + bh2_ref[0:1, :])
        z3 = _relu(jnp.dot(z2, wh3_ref[...], preferred_element_type=F32)
                   + bh3_ref[0:1, :])
        z4 = jnp.sum(z3 * wh4_ref[0:1, :], axis=1, keepdims=True) \
            + misc_ref[0:1, 0:1]
        rid = lax.broadcasted_iota(jnp.int32, (B, 1), 0) + pid * B
        z4 = jnp.where(rid < e_count, z4, 0.0)
        iog = lax.broadcasted_iota(jnp.int32, (1, g), 1).astype(F32)
        onehot = (eb_ref[...] == iog).astype(F32)
        pooled = lax.dot_general(z4, onehot, (((0,), (0,)), ((), ())),
                                 preferred_element_type=F32)

        @pl.when(pid == 0)
        def _():
            out_ref[...] = jnp.zeros((8, g), F32)

        out_ref[...] = out_ref[...] + jnp.concatenate(
            [pooled, jnp.zeros((7, g), F32)], axis=0)

    full = lambda i: (0, 0)
    return pl.pallas_call(
        body,
        grid=(grid,),
        in_specs=[
            pl.BlockSpec((B, 128), lambda i: (i, 0)),
            pl.BlockSpec((B, 128), lambda i: (i, 0)),
            pl.BlockSpec((B, 6), lambda i: (i, 0)),
            pl.BlockSpec((B, 1), lambda i: (i, 0)),
            pl.BlockSpec((8, 128), full),
            pl.BlockSpec((ef, 22), full),
            pl.BlockSpec((8, 22), full),
            pl.BlockSpec((22, 40), full),
            pl.BlockSpec((8, 40), full),
            pl.BlockSpec((40, 40), full),
            pl.BlockSpec((8, 40), full),
            pl.BlockSpec((128, 512), full),
            pl.BlockSpec((128, 512), full),
            pl.BlockSpec((40, 512), full),
            pl.BlockSpec((8, 512), full),
            pl.BlockSpec((512, 512), full),
            pl.BlockSpec((8, 512), full),
            pl.BlockSpec((512, 256), full),
            pl.BlockSpec((8, 256), full),
            pl.BlockSpec((8, 256), full),
            pl.BlockSpec((8, 8), full),
        ],
        out_specs=pl.BlockSpec((8, g), full),
        out_shape=jax.ShapeDtypeStruct((8, g), F32),
        compiler_params=pltpu.CompilerParams(dimension_semantics=("arbitrary",)),
    )(gs, gd, eap, ebp, ST2, M1, be1, We2, be2, We3, be3,
      Wh1a, Wh1b, Wh1c, bh1, Wh2, bh2, Wh3, bh3, wh4, misc)


def _pad_bias(b, n):
    out = jnp.zeros((8, n), F32)
    return out.at[0, :b.shape[0]].set(b)


def kernel(x, edge_index, edge_attr, edge_batch, pos, params):
    n = x.shape[0]
    e = edge_index.shape[1]
    g = 64
    npad = -(-n // 1024) * 1024
    epad = -(-e // (NC * NS * CH)) * (NC * NS * CH)

    node_tabs = params['node_emb']
    edge_tabs = params['edge_emb']
    nv = [t.shape[0] for t in node_tabs]
    nd = [t.shape[1] for t in node_tabs]
    ev = [t.shape[0] for t in edge_tabs]
    ed = [t.shape[1] for t in edge_tabs]
    n_voff = [0]
    for v in nv:
        n_voff.append(n_voff[-1] + v)
    n_doff = [0]
    for dd in nd:
        n_doff.append(n_doff[-1] + dd)
    e_voff = [0]
    for v in ev:
        e_voff.append(e_voff[-1] + v)
    e_doff = [0]
    for dd in ed:
        e_doff.append(e_doff[-1] + dd)
    nf = -(-(n_voff[-1] + 4) // 8) * 8
    ef = -(-(e_voff[-1] + 1) // 8) * 8

    T0 = jnp.zeros((nf, 128), F32)
    for i in range(9):
        T0 = T0.at[n_voff[i]:n_voff[i] + nv[i],
                   n_doff[i]:n_doff[i] + nd[i]].set(node_tabs[i])
    T0 = T0.at[n_voff[-1], n_doff[-1]].set(1.0)
    for j in range(3):
        T0 = T0.at[n_voff[-1] + 1 + j, n_doff[-1] + 1 + j].set(0.1)
    node_onehot_off = n_voff[:9] + [n_voff[-1]]
    edge_onehot_off = e_voff[:5] + [e_voff[-1]]

    ea_map = jnp.zeros((ef, 11), F32)
    for i in range(5):
        ea_map = ea_map.at[e_voff[i]:e_voff[i] + ev[i],
                           e_doff[i]:e_doff[i] + ed[i]].set(edge_tabs[i])
    ea_map = ea_map.at[e_voff[-1], 10].set(0.1)
    M1 = ea_map @ params['e_lin'][0]['w'].T

    W1 = jnp.zeros((128, 108), F32).at[:37].set(params['x_nn1'][0]['w'].T)
    b1 = _pad_bias(params['x_nn1'][0]['b'], 108)
    W2f = jnp.zeros((108, 128), F32).at[:, :108].set(params['x_nn1'][1]['w'].T)
    b2f = _pad_bias(params['x_nn1'][1]['b'], 128).at[0, 108].set(1.0)
    W3 = jnp.zeros((128, 108), F32).at[:108].set(params['x_nn2'][0]['w'].T)
    b3 = _pad_bias(params['x_nn2'][0]['b'], 108)
    W4f = jnp.zeros((108, 128), F32).at[:, :108].set(params['x_nn2'][1]['w'].T)
    b4f = _pad_bias(params['x_nn2'][1]['b'], 128)

    h0w = params['head'][0]['w']
    Wh1a = jnp.zeros((128, 512), F32).at[:108].set(h0w[:, :108].T)
    Wh1b = jnp.zeros((128, 512), F32).at[:108].set(h0w[:, 108:216].T)
    Wh1c = h0w[:, 216:256].T
    bh1 = _pad_bias(params['head'][0]['b'], 512)
    Wh2 = params['head'][1]['w'].T
    bh2 = _pad_bias(params['head'][1]['b'], 512)
    Wh3 = params['head'][2]['w'].T
    bh3 = _pad_bias(params['head'][2]['b'], 256)
    wh4 = jnp.zeros((8, 256), F32).at[0].set(params['head'][3]['w'][0])
    misc = jnp.zeros((8, 8), F32).at[0, 0].set(params['head'][3]['b'][0])
    be1 = _pad_bias(params['e_lin'][0]['b'], 22)
    We2 = params['e_lin'][1]['w'].T
    be2 = _pad_bias(params['e_lin'][1]['b'], 40)
    We3 = params['e_lin'][2]['w'].T
    be3 = _pad_bias(params['e_lin'][2]['b'], 40)

    xp = jnp.zeros((npad, 13), F32).at[:n].set(x)
    posp = jnp.zeros((npad, 3), F32).at[:n].set(pos)
    srcp = jnp.full((epad,), n, jnp.int32).at[:e].set(edge_index[0])
    dstp = jnp.full((epad,), n, jnp.int32).at[:e].set(edge_index[1])
    src2 = srcp.reshape(-1, CW)
    dst2 = dstp.reshape(-1, CW)
    src2s = srcp.reshape(-1, CWS)
    dst2s = dstp.reshape(-1, CWS)

    eap = jnp.zeros((epad, 6), F32).at[:e].set(edge_attr)
    ebp = jnp.zeros((epad, 1), F32).at[:e, 0].set(edge_batch.astype(F32))

    h0 = _tc_node_features(xp, posp, T0, npad, nf, node_onehot_off)
    agg0 = _sc_scatter_add(h0, src2s, dst2s, npad, epad)
    dummy_st = jnp.zeros((8, 128), F32)
    u1, st1 = _tc_gin(h0, agg0[:npad], agg0[npad:], W1, b1, W2f, b2f,
                      dummy_st, npad, n, fold_bn=False)
    mean1 = st1[0, :108] / n
    var1 = st1[1, :108] / n - mean1 * mean1
    s1 = params['bn1']['g'] / jnp.sqrt(var1 + 1e-5)
    t1 = params['bn1']['b'] - mean1 * s1
    ST1 = jnp.zeros((8, 128), F32).at[0, :108].set(s1).at[1, :108].set(t1)

    aggu = _sc_scatter_add(u1, src2s, dst2s, npad, epad)
    u2, st2 = _tc_gin(u1, aggu[:npad], aggu[npad:], W3, b3, W4f, b4f,
                      ST1, npad, n, fold_bn=True)
    mean2 = st2[0, :108] / n
    var2 = st2[1, :108] / n - mean2 * mean2
    s2 = params['bn2']['g'] / jnp.sqrt(var2 + 1e-5)
    t2 = params['bn2']['b'] - mean2 * s2
    ST2 = jnp.zeros((8, 128), F32).at[0, :108].set(s2).at[1, :108].set(t2)

    gs, gd = _sc_gather2(u2, src2, dst2, epad)
    pooled = _tc_head(gs, gd, eap, ebp, ST2, M1, be1, We2, be2, We3, be3,
                      Wh1a, Wh1b, Wh1c, bh1, Wh2, bh2, Wh3, bh3, wh4, misc,
                      epad, e, ef, edge_onehot_off, g)
    return pooled[0, :].reshape(g, 1)


# speedup vs baseline: 1.5395x; 1.0233x over previous
import functools

import jax
import jax.numpy as jnp
from jax import lax
from jax.experimental import pallas as pl
from jax.experimental.pallas import tpu as pltpu
from jax.experimental.pallas import tpu_sc as plsc

F32 = jnp.float32

NC = 2
NS = 16
CH = 128


def _relu(v):
    return jnp.maximum(v, 0.0)


def _tc_node_features(xp, posp, T0, npad, nf, n_off):
    B = 1024
    grid = npad // B

    def body(x_ref, pos_ref, t_ref, o_ref):
        xb = x_ref[...]
        iota = lax.broadcasted_iota(jnp.int32, (1, nf), 1)
        acc = jnp.zeros((B, nf), F32)
        for i in range(9):
            gi = xb[:, i:i + 1].astype(jnp.int32) + n_off[i]
            acc = acc + (gi == iota).astype(F32)
        acc = acc + xb[:, 9:10] * (iota == n_off[9]).astype(F32)
        pb = pos_ref[...]
        for j in range(3):
            acc = acc + pb[:, j:j + 1] * (iota == (n_off[9] + 1 + j)).astype(F32)
        o_ref[...] = jnp.dot(acc, t_ref[...], preferred_element_type=F32)

    return pl.pallas_call(
        body,
        grid=(grid,),
        in_specs=[
            pl.BlockSpec((B, 13), lambda i: (i, 0)),
            pl.BlockSpec((B, 3), lambda i: (i, 0)),
            pl.BlockSpec((nf, 128), lambda i: (0, 0)),
        ],
        out_specs=pl.BlockSpec((B, 128), lambda i: (i, 0)),
        out_shape=jax.ShapeDtypeStruct((npad, 128), F32),
        compiler_params=pltpu.CompilerParams(dimension_semantics=("arbitrary",)),
    )(xp, posp, T0)


NB = 8
CW = 64


_SC_CACHE = {}


CWS = 128


def _sc_scatter_add(h, src2s, dst2s, npad, epad):
    zeros = jnp.zeros((npad // NS, 128), F32)
    key = ("scatter", npad, epad)
    if key not in _SC_CACHE:
        _SC_CACHE[key] = _build_sc_scatter(npad, epad)
    return _SC_CACHE[key](zeros, h, src2s, dst2s)


def _build_sc_scatter(npad, epad):
    nrows = epad // CWS
    nch = nrows // (NC * NS)
    rps = npad // NS
    mesh = plsc.VectorSubcoreMesh(core_axis_name="c", subcore_axis_name="s")

    @functools.partial(
        pl.kernel,
        out_type=jax.ShapeDtypeStruct((NC * npad, 128), F32),
        mesh=mesh,
        scratch_types=[
            pltpu.VMEM((nch, CWS), jnp.int32),
            pltpu.VMEM((nch, CWS), jnp.int32),
            pltpu.VMEM((CWS, 128), F32),
            pltpu.VMEM((CWS, 128), F32),
            pltpu.VMEM_SHARED((npad, 128), F32),
            pltpu.SemaphoreType.DMA,
            pltpu.SemaphoreType.DMA,
        ],
    )
    def k(zeros_hbm, h_hbm, src_hbm, dst_hbm, out_hbm, sidx, didx, rows0,
          rows1, acc, sem0, sem1):
        c = lax.axis_index("c")
        s = lax.axis_index("s")
        wid = c * NS + s
        crow = wid * nch
        r0 = s * rps
        pltpu.sync_copy(src_hbm.at[pl.ds(crow, nch)], sidx)
        pltpu.sync_copy(dst_hbm.at[pl.ds(crow, nch)], didx)
        pltpu.sync_copy(zeros_hbm, acc.at[pl.ds(r0, rps)])
        plsc.subcore_barrier()

        pltpu.async_copy(h_hbm.at[sidx.at[0]], rows0, sem0)

        def step(i, carry):
            j0 = 2 * i
            pltpu.async_copy(h_hbm.at[sidx.at[j0 + 1]], rows1, sem1)
            pltpu.make_async_copy(h_hbm.at[sidx.at[j0]], rows0, sem0).wait()
            pltpu.sync_copy(rows0, acc.at[didx.at[j0]], add=True)

            @pl.when(j0 + 2 < nch)
            def _():
                pltpu.async_copy(h_hbm.at[sidx.at[j0 + 2]], rows0, sem0)

            pltpu.make_async_copy(h_hbm.at[sidx.at[j0 + 1]], rows1,
                                  sem1).wait()
            pltpu.sync_copy(rows1, acc.at[didx.at[j0 + 1]], add=True)
            return carry

        lax.fori_loop(0, nch // 2, step, 0)
        plsc.subcore_barrier()
        pltpu.sync_copy(acc.at[pl.ds(r0, rps)],
                        out_hbm.at[pl.ds(c * npad + r0, rps)])

    return k


def _sc_gather2(u, src2, dst2, epad):
    nch = epad // CW // (NC * NS)
    mesh = plsc.VectorSubcoreMesh(core_axis_name="c", subcore_axis_name="s")
    sds = jax.ShapeDtypeStruct((epad, 128), F32)
    NP = NB // 2

    @functools.partial(
        pl.kernel,
        out_type=(sds, sds),
        mesh=mesh,
        scratch_types=[
            pltpu.VMEM((nch, CW), jnp.int32),
            pltpu.VMEM((nch, CW), jnp.int32),
            [pltpu.VMEM((CW, 128), F32)] * NB,
            [pltpu.SemaphoreType.DMA] * NB,
            [pltpu.SemaphoreType.DMA] * NB,
        ],
    )
    def k(u_hbm, src_hbm, dst_hbm, gs_hbm, gd_hbm, sidx, didx, rows,
          gsem, wsem):
        c = lax.axis_index("c")
        s = lax.axis_index("s")
        wid = c * NS + s
        crow = wid * nch
        base = crow * CW
        pltpu.sync_copy(src_hbm.at[pl.ds(crow, nch)], sidx)
        pltpu.sync_copy(dst_hbm.at[pl.ds(crow, nch)], didx)

        idxs = [sidx] * NP + [didx] * NP

        def fire(b, j):
            pltpu.async_copy(u_hbm.at[idxs[b].at[j]], rows[b], gsem[b])

        for b in range(NB):
            fire(b, b % NP)

        def step(i, carry):
            for b in range(NB):
                p = b % NP
                j = i * NP + p
                out = gs_hbm if b < NP else gd_hbm
                pltpu.make_async_copy(u_hbm.at[idxs[b].at[j]], rows[b],
                                      gsem[b]).wait()
                pltpu.async_copy(rows[b], out.at[pl.ds(base + j * CW, CW)],
                                 wsem[b])

                @pl.when(j + NP < nch)
                def _():
                    pltpu.make_async_copy(
                        rows[b], out.at[pl.ds(base + j * CW, CW)],
                        wsem[b]).wait()
                    fire(b, j + NP)
            return carry

        lax.fori_loop(0, nch // NP, step, 0)
        for b in range(NB):
            p = b % NP
            out = gs_hbm if b < NP else gd_hbm
            pltpu.make_async_copy(
                rows[b], out.at[pl.ds(base + (nch - NP + p) * CW, CW)],
                wsem[b]).wait()

    return k(u, src2, dst2)


def _tc_gin(hin, agg_a, agg_b, W1, b1, W2f, b2f, ST, npad, n, fold_bn):
    B = 1024
    grid = npad // B

    def body(h_ref, aa_ref, ab_ref, w1_ref, b1_ref, w2_ref, b2_ref, st_ref,
             u_ref, stat_ref):
        pid = pl.program_id(0)
        h = h_ref[...]
        ag = aa_ref[...] + ab_ref[...]
        if fold_bn:
            S = st_ref[0:1, :]
            T = st_ref[1:2, :]
            onehot108 = (lax.broadcasted_iota(jnp.int32, (1, 128), 1) == 108)
            indeg = jnp.sum(jnp.where(onehot108, ag, 0.0), axis=1, keepdims=True)
            z = h * S + T + S * ag + T * indeg
        else:
            z = h + ag
        t = _relu(jnp.dot(z, w1_ref[...], preferred_element_type=F32)
                  + b1_ref[0:1, :])
        u = _relu(jnp.dot(t, w2_ref[...], preferred_element_type=F32)
                  + b2_ref[0:1, :])
        u_ref[...] = u
        rid = lax.broadcasted_iota(jnp.int32, (B, 1), 0) + pid * B
        um = jnp.where(rid < n, u, 0.0)
        ssum = jnp.sum(um, axis=0, keepdims=True)
        sq = jnp.sum(um * um, axis=0, keepdims=True)

        @pl.when(pid == 0)
        def _():
            stat_ref[...] = jnp.zeros((8, 128), F32)

        upd = jnp.concatenate([ssum, sq, jnp.zeros((6, 128), F32)], axis=0)
        stat_ref[...] = stat_ref[...] + upd

    return pl.pallas_call(
        body,
        grid=(grid,),
        in_specs=[
            pl.BlockSpec((B, 128), lambda i: (i, 0)),
            pl.BlockSpec((B, 128), lambda i: (i, 0)),
            pl.BlockSpec((B, 128), lambda i: (i, 0)),
            pl.BlockSpec((128, 108), lambda i: (0, 0)),
            pl.BlockSpec((8, 108), lambda i: (0, 0)),
            pl.BlockSpec((108, 128), lambda i: (0, 0)),
            pl.BlockSpec((8, 128), lambda i: (0, 0)),
            pl.BlockSpec((8, 128), lambda i: (0, 0)),
        ],
        out_specs=[
            pl.BlockSpec((B, 128), lambda i: (i, 0)),
            pl.BlockSpec((8, 128), lambda i: (0, 0)),
        ],
        out_shape=[
            jax.ShapeDtypeStruct((npad, 128), F32),
            jax.ShapeDtypeStruct((8, 128), F32),
        ],
        compiler_params=pltpu.CompilerParams(dimension_semantics=("arbitrary",)),
    )(hin, agg_a, agg_b, W1, b1, W2f, b2f, ST)


def _tc_head(gs, gd, eap, ebp, ST2, M1, be1, We2, be2, We3, be3,
             Wh1a, Wh1b, Wh1c, bh1, Wh2, bh2, Wh3, bh3, wh4, misc,
             epad, e_count, ef, e_off, g):
    B = 512
    grid = epad // B

    def body(gs_ref, gd_ref, ea_ref, eb_ref, st_ref, m1_ref, be1_ref,
             we2_ref, be2_ref, we3_ref, be3_ref, wa_ref, wb_ref, wc_ref,
             bh1_ref, wh2_ref, bh2_ref, wh3_ref, bh3_ref, wh4_ref, misc_ref,
             out_ref):
        pid = pl.program_id(0)
        ea = ea_ref[...]
        iote = lax.broadcasted_iota(jnp.int32, (1, ef), 1)
        acc = jnp.zeros((B, ef), F32)
        for i in range(5):
            gi = ea[:, i:i + 1].astype(jnp.int32) + e_off[i]
            acc = acc + (gi == iote).astype(F32)
        acc = acc + ea[:, 5:6] * (iote == e_off[5]).astype(F32)
        e1 = _relu(jnp.dot(acc, m1_ref[...], preferred_element_type=F32)
                   + be1_ref[0:1, :])
        e2 = _relu(jnp.dot(e1, we2_ref[...], preferred_element_type=F32)
                   + be2_ref[0:1, :])
        e3 = jnp.dot(e2, we3_ref[...], preferred_element_type=F32) + be3_ref[0:1, :]
        S = st_ref[0:1, :]
        T = st_ref[1:2, :]
        xs = gs_ref[...] * S + T
        xd = gd_ref[...] * S + T
        z1 = _relu(jnp.dot(xs, wa_ref[...], preferred_element_type=F32)
                   + jnp.dot(xd, wb_ref[...], preferred_element_type=F32)
                   + jnp.dot(e3, wc_ref[...], preferred_element_type=F32)
                   + bh1_ref[0:1, :])
        z2 = _relu(jnp.dot(z1, wh2_ref[...], preferred_element_type=F32)
                   + bh2_ref[0:1, :])
        z3 = _relu(jnp.dot(z2, wh3_ref[...], preferred_element_type=F32)
                   + bh3_ref[0:1, :])
        z4 = jnp.sum(z3 * wh4_ref[0:1, :], axis=1, keepdims=True) \
            + misc_ref[0:1, 0:1]
        rid = lax.broadcasted_iota(jnp.int32, (B, 1), 0) + pid * B
        z4 = jnp.where(rid < e_count, z4, 0.0)
        iog = lax.broadcasted_iota(jnp.int32, (1, g), 1).astype(F32)
        onehot = (eb_ref[...] == iog).astype(F32)
        pooled = lax.dot_general(z4, onehot, (((0,), (0,)), ((), ())),
                                 preferred_element_type=F32)

        @pl.when(pid == 0)
        def _():
            out_ref[...] = jnp.zeros((8, g), F32)

        out_ref[...] = out_ref[...] + jnp.concatenate(
            [pooled, jnp.zeros((7, g), F32)], axis=0)

    full = lambda i: (0, 0)
    return pl.pallas_call(
        body,
        grid=(grid,),
        in_specs=[
            pl.BlockSpec((B, 128), lambda i: (i, 0)),
            pl.BlockSpec((B, 128), lambda i: (i, 0)),
            pl.BlockSpec((B, 6), lambda i: (i, 0)),
            pl.BlockSpec((B, 1), lambda i: (i, 0)),
            pl.BlockSpec((8, 128), full),
            pl.BlockSpec((ef, 22), full),
            pl.BlockSpec((8, 22), full),
            pl.BlockSpec((22, 40), full),
            pl.BlockSpec((8, 40), full),
            pl.BlockSpec((40, 40), full),
            pl.BlockSpec((8, 40), full),
            pl.BlockSpec((128, 512), full),
            pl.BlockSpec((128, 512), full),
            pl.BlockSpec((40, 512), full),
            pl.BlockSpec((8, 512), full),
            pl.BlockSpec((512, 512), full),
            pl.BlockSpec((8, 512), full),
            pl.BlockSpec((512, 256), full),
            pl.BlockSpec((8, 256), full),
            pl.BlockSpec((8, 256), full),
            pl.BlockSpec((8, 8), full),
        ],
        out_specs=pl.BlockSpec((8, g), full),
        out_shape=jax.ShapeDtypeStruct((8, g), F32),
        compiler_params=pltpu.CompilerParams(dimension_semantics=("arbitrary",)),
    )(gs, gd, eap, ebp, ST2, M1, be1, We2, be2, We3, be3,
      Wh1a, Wh1b, Wh1c, bh1, Wh2, bh2, Wh3, bh3, wh4, misc)


def _pad_bias(b, n):
    out = jnp.zeros((8, n), F32)
    return out.at[0, :b.shape[0]].set(b)


def kernel(x, edge_index, edge_attr, edge_batch, pos, params):
    n = x.shape[0]
    e = edge_index.shape[1]
    g = 64
    npad = -(-n // 1024) * 1024
    epad = -(-e // (NC * NS * CH)) * (NC * NS * CH)

    node_tabs = params['node_emb']
    edge_tabs = params['edge_emb']
    nv = [t.shape[0] for t in node_tabs]
    nd = [t.shape[1] for t in node_tabs]
    ev = [t.shape[0] for t in edge_tabs]
    ed = [t.shape[1] for t in edge_tabs]
    n_voff = [0]
    for v in nv:
        n_voff.append(n_voff[-1] + v)
    n_doff = [0]
    for dd in nd:
        n_doff.append(n_doff[-1] + dd)
    e_voff = [0]
    for v in ev:
        e_voff.append(e_voff[-1] + v)
    e_doff = [0]
    for dd in ed:
        e_doff.append(e_doff[-1] + dd)
    nf = -(-(n_voff[-1] + 4) // 8) * 8
    ef = -(-(e_voff[-1] + 1) // 8) * 8

    T0 = jnp.zeros((nf, 128), F32)
    for i in range(9):
        T0 = T0.at[n_voff[i]:n_voff[i] + nv[i],
                   n_doff[i]:n_doff[i] + nd[i]].set(node_tabs[i])
    T0 = T0.at[n_voff[-1], n_doff[-1]].set(1.0)
    for j in range(3):
        T0 = T0.at[n_voff[-1] + 1 + j, n_doff[-1] + 1 + j].set(0.1)
    node_onehot_off = n_voff[:9] + [n_voff[-1]]
    edge_onehot_off = e_voff[:5] + [e_voff[-1]]

    ea_map = jnp.zeros((ef, 11), F32)
    for i in range(5):
        ea_map = ea_map.at[e_voff[i]:e_voff[i] + ev[i],
                           e_doff[i]:e_doff[i] + ed[i]].set(edge_tabs[i])
    ea_map = ea_map.at[e_voff[-1], 10].set(0.1)
    M1 = ea_map @ params['e_lin'][0]['w'].T

    W1 = jnp.zeros((128, 108), F32).at[:37].set(params['x_nn1'][0]['w'].T)
    b1 = _pad_bias(params['x_nn1'][0]['b'], 108)
    W2f = jnp.zeros((108, 128), F32).at[:, :108].set(params['x_nn1'][1]['w'].T)
    b2f = _pad_bias(params['x_nn1'][1]['b'], 128).at[0, 108].set(1.0)
    W3 = jnp.zeros((128, 108), F32).at[:108].set(params['x_nn2'][0]['w'].T)
    b3 = _pad_bias(params['x_nn2'][0]['b'], 108)
    W4f = jnp.zeros((108, 128), F32).at[:, :108].set(params['x_nn2'][1]['w'].T)
    b4f = _pad_bias(params['x_nn2'][1]['b'], 128)

    h0w = params['head'][0]['w']
    Wh1a = jnp.zeros((128, 512), F32).at[:108].set(h0w[:, :108].T)
    Wh1b = jnp.zeros((128, 512), F32).at[:108].set(h0w[:, 108:216].T)
    Wh1c = h0w[:, 216:256].T
    bh1 = _pad_bias(params['head'][0]['b'], 512)
    Wh2 = params['head'][1]['w'].T
    bh2 = _pad_bias(params['head'][1]['b'], 512)
    Wh3 = params['head'][2]['w'].T
    bh3 = _pad_bias(params['head'][2]['b'], 256)
    wh4 = jnp.zeros((8, 256), F32).at[0].set(params['head'][3]['w'][0])
    misc = jnp.zeros((8, 8), F32).at[0, 0].set(params['head'][3]['b'][0])
    be1 = _pad_bias(params['e_lin'][0]['b'], 22)
    We2 = params['e_lin'][1]['w'].T
    be2 = _pad_bias(params['e_lin'][1]['b'], 40)
    We3 = params['e_lin'][2]['w'].T
    be3 = _pad_bias(params['e_lin'][2]['b'], 40)

    xp = jnp.zeros((npad, 13), F32).at[:n].set(x)
    posp = jnp.zeros((npad, 3), F32).at[:n].set(pos)
    srcp = jnp.full((epad,), n, jnp.int32).at[:e].set(edge_index[0])
    dstp = jnp.full((epad,), n, jnp.int32).at[:e].set(edge_index[1])
    src2 = srcp.reshape(-1, CW)
    dst2 = dstp.reshape(-1, CW)
    src2s = srcp.reshape(-1, CWS)
    dst2s = dstp.reshape(-1, CWS)

    eap = jnp.zeros((epad, 6), F32).at[:e].set(edge_attr)
    ebp = jnp.zeros((epad, 1), F32).at[:e, 0].set(edge_batch.astype(F32))

    h0 = _tc_node_features(xp, posp, T0, npad, nf, node_onehot_off)
    agg0 = _sc_scatter_add(h0, src2s, dst2s, npad, epad)
    dummy_st = jnp.zeros((8, 128), F32)
    u1, st1 = _tc_gin(h0, agg0[:npad], agg0[npad:], W1, b1, W2f, b2f,
                      dummy_st, npad, n, fold_bn=False)
    mean1 = st1[0, :108] / n
    var1 = st1[1, :108] / n - mean1 * mean1
    s1 = params['bn1']['g'] / jnp.sqrt(var1 + 1e-5)
    t1 = params['bn1']['b'] - mean1 * s1
    ST1 = jnp.zeros((8, 128), F32).at[0, :108].set(s1).at[1, :108].set(t1)

    aggu = _sc_scatter_add(u1, src2s, dst2s, npad, epad)
    u2, st2 = _tc_gin(u1, aggu[:npad], aggu[npad:], W3, b3, W4f, b4f,
                      ST1, npad, n, fold_bn=True)
    mean2 = st2[0, :108] / n
    var2 = st2[1, :108] / n - mean2 * mean2
    s2 = params['bn2']['g'] / jnp.sqrt(var2 + 1e-5)
    t2 = params['bn2']['b'] - mean2 * s2
    ST2 = jnp.zeros((8, 128), F32).at[0, :108].set(s2).at[1, :108].set(t2)

    gs, gd = _sc_gather2(u2, src2, dst2, epad)
    pooled = _tc_head(gs, gd, eap, ebp, ST2, M1, be1, We2, be2, We3, be3,
                      Wh1a, Wh1b, Wh1c, bh1, Wh2, bh2, Wh3, bh3, wh4, misc,
                      epad, e, ef, edge_onehot_off, g)
    return pooled[0, :].reshape(g, 1)


# speedup vs baseline: 1.5547x; 1.0099x over previous
import functools

import jax
import jax.numpy as jnp
from jax import lax
from jax.experimental import pallas as pl
from jax.experimental.pallas import tpu as pltpu
from jax.experimental.pallas import tpu_sc as plsc

F32 = jnp.float32

NC = 2
NS = 16
CH = 128


def _relu(v):
    return jnp.maximum(v, 0.0)


def _tc_node_features(xp, posp, T0, npad, nf, n_off):
    B = 1024
    grid = npad // B

    def body(x_ref, pos_ref, t_ref, o_ref):
        xb = x_ref[...]
        iota = lax.broadcasted_iota(jnp.int32, (1, nf), 1)
        acc = jnp.zeros((B, nf), F32)
        for i in range(9):
            gi = xb[:, i:i + 1].astype(jnp.int32) + n_off[i]
            acc = acc + (gi == iota).astype(F32)
        acc = acc + xb[:, 9:10] * (iota == n_off[9]).astype(F32)
        pb = pos_ref[...]
        for j in range(3):
            acc = acc + pb[:, j:j + 1] * (iota == (n_off[9] + 1 + j)).astype(F32)
        o_ref[...] = jnp.dot(acc, t_ref[...], preferred_element_type=F32)

    return pl.pallas_call(
        body,
        grid=(grid,),
        in_specs=[
            pl.BlockSpec((B, 13), lambda i: (i, 0)),
            pl.BlockSpec((B, 3), lambda i: (i, 0)),
            pl.BlockSpec((nf, 128), lambda i: (0, 0)),
        ],
        out_specs=pl.BlockSpec((B, 128), lambda i: (i, 0)),
        out_shape=jax.ShapeDtypeStruct((npad, 128), F32),
        compiler_params=pltpu.CompilerParams(dimension_semantics=("arbitrary",)),
    )(xp, posp, T0)


NB = 8
CW = 64


_SC_CACHE = {}


CWS = 128


def _sc_scatter_add(h, src2s, dst2s, npad, epad):
    zeros = jnp.zeros((npad // NS, 128), F32)
    key = ("scatter", npad, epad)
    if key not in _SC_CACHE:
        _SC_CACHE[key] = _build_sc_scatter(npad, epad)
    return _SC_CACHE[key](zeros, h, src2s, dst2s)


def _build_sc_scatter(npad, epad):
    nrows = epad // CWS
    nch = nrows // (NC * NS)
    rps = npad // NS
    mesh = plsc.VectorSubcoreMesh(core_axis_name="c", subcore_axis_name="s")

    @functools.partial(
        pl.kernel,
        out_type=jax.ShapeDtypeStruct((NC * npad, 128), F32),
        mesh=mesh,
        scratch_types=[
            pltpu.VMEM((nch, CWS), jnp.int32),
            pltpu.VMEM((nch, CWS), jnp.int32),
            pltpu.VMEM((CWS, 128), F32),
            pltpu.VMEM((CWS, 128), F32),
            pltpu.VMEM_SHARED((npad, 128), F32),
            pltpu.SemaphoreType.DMA,
            pltpu.SemaphoreType.DMA,
        ],
    )
    def k(zeros_hbm, h_hbm, src_hbm, dst_hbm, out_hbm, sidx, didx, rows0,
          rows1, acc, sem0, sem1):
        rowsl = (rows0, rows1)
        seml = (sem0, sem1)
        c = lax.axis_index("c")
        s = lax.axis_index("s")
        wid = c * NS + s
        crow = wid * nch
        r0 = s * rps
        pltpu.sync_copy(src_hbm.at[pl.ds(crow, nch)], sidx)
        pltpu.sync_copy(dst_hbm.at[pl.ds(crow, nch)], didx)
        pltpu.sync_copy(zeros_hbm, acc.at[pl.ds(r0, rps)])
        plsc.subcore_barrier()

        for b in range(2):
            pltpu.async_copy(h_hbm.at[sidx.at[b]], rowsl[b], seml[b])

        def step(i, carry):
            for b in range(2):
                j = 2 * i + b
                pltpu.make_async_copy(h_hbm.at[sidx.at[j]], rowsl[b],
                                      seml[b]).wait()
                pltpu.sync_copy(rowsl[b], acc.at[didx.at[j]], add=True)

                @pl.when(j + 2 < nch)
                def _():
                    pltpu.async_copy(h_hbm.at[sidx.at[j + 2]], rowsl[b],
                                     seml[b])
            return carry

        lax.fori_loop(0, nch // 2, step, 0)
        plsc.subcore_barrier()
        pltpu.sync_copy(acc.at[pl.ds(r0, rps)],
                        out_hbm.at[pl.ds(c * npad + r0, rps)])

    return k


def _sc_gather2(u, src2s, dst2s, epad):
    nch = epad // CWS // (NC * NS)
    mesh = plsc.VectorSubcoreMesh(core_axis_name="c", subcore_axis_name="s")
    sds = jax.ShapeDtypeStruct((epad, 128), F32)
    GNB = 4
    NP = 2

    @functools.partial(
        pl.kernel,
        out_type=(sds, sds),
        mesh=mesh,
        scratch_types=[
            pltpu.VMEM((nch, CWS), jnp.int32),
            pltpu.VMEM((nch, CWS), jnp.int32),
            [pltpu.VMEM((CWS, 128), F32)] * GNB,
            [pltpu.SemaphoreType.DMA] * GNB,
            [pltpu.SemaphoreType.DMA] * GNB,
        ],
    )
    def k(u_hbm, src_hbm, dst_hbm, gs_hbm, gd_hbm, sidx, didx, rows,
          gsem, wsem):
        c = lax.axis_index("c")
        s = lax.axis_index("s")
        wid = c * NS + s
        crow = wid * nch
        base = crow * CWS
        pltpu.sync_copy(src_hbm.at[pl.ds(crow, nch)], sidx)
        pltpu.sync_copy(dst_hbm.at[pl.ds(crow, nch)], didx)

        idxs = [sidx] * NP + [didx] * NP

        def fire(b, j):
            pltpu.async_copy(u_hbm.at[idxs[b].at[j]], rows[b], gsem[b])

        for b in range(GNB):
            fire(b, b % NP)

        def step(i, carry):
            for b in range(GNB):
                p = b % NP
                j = i * NP + p
                out = gs_hbm if b < NP else gd_hbm
                pltpu.make_async_copy(u_hbm.at[idxs[b].at[j]], rows[b],
                                      gsem[b]).wait()
                pltpu.async_copy(rows[b], out.at[pl.ds(base + j * CWS, CWS)],
                                 wsem[b])

                @pl.when(j + NP < nch)
                def _():
                    pltpu.make_async_copy(
                        rows[b], out.at[pl.ds(base + j * CWS, CWS)],
                        wsem[b]).wait()
                    fire(b, j + NP)
            return carry

        lax.fori_loop(0, nch // NP, step, 0)
        for b in range(GNB):
            p = b % NP
            out = gs_hbm if b < NP else gd_hbm
            pltpu.make_async_copy(
                rows[b], out.at[pl.ds(base + (nch - NP + p) * CWS, CWS)],
                wsem[b]).wait()

    return k(u, src2s, dst2s)


def _tc_gin(hin, agg_a, agg_b, W1, b1, W2f, b2f, ST, npad, n, fold_bn):
    B = 1024
    grid = npad // B

    def body(h_ref, aa_ref, ab_ref, w1_ref, b1_ref, w2_ref, b2_ref, st_ref,
             u_ref, stat_ref):
        pid = pl.program_id(0)
        h = h_ref[...]
        ag = aa_ref[...] + ab_ref[...]
        if fold_bn:
            S = st_ref[0:1, :]
            T = st_ref[1:2, :]
            onehot108 = (lax.broadcasted_iota(jnp.int32, (1, 128), 1) == 108)
            indeg = jnp.sum(jnp.where(onehot108, ag, 0.0), axis=1, keepdims=True)
            z = h * S + T + S * ag + T * indeg
        else:
            z = h + ag
        t = _relu(jnp.dot(z, w1_ref[...], preferred_element_type=F32)
                  + b1_ref[0:1, :])
        u = _relu(jnp.dot(t, w2_ref[...], preferred_element_type=F32)
                  + b2_ref[0:1, :])
        u_ref[...] = u
        rid = lax.broadcasted_iota(jnp.int32, (B, 1), 0) + pid * B
        um = jnp.where(rid < n, u, 0.0)
        ssum = jnp.sum(um, axis=0, keepdims=True)
        sq = jnp.sum(um * um, axis=0, keepdims=True)

        @pl.when(pid == 0)
        def _():
            stat_ref[...] = jnp.zeros((8, 128), F32)

        upd = jnp.concatenate([ssum, sq, jnp.zeros((6, 128), F32)], axis=0)
        stat_ref[...] = stat_ref[...] + upd

    return pl.pallas_call(
        body,
        grid=(grid,),
        in_specs=[
            pl.BlockSpec((B, 128), lambda i: (i, 0)),
            pl.BlockSpec((B, 128), lambda i: (i, 0)),
            pl.BlockSpec((B, 128), lambda i: (i, 0)),
            pl.BlockSpec((128, 108), lambda i: (0, 0)),
            pl.BlockSpec((8, 108), lambda i: (0, 0)),
            pl.BlockSpec((108, 128), lambda i: (0, 0)),
            pl.BlockSpec((8, 128), lambda i: (0, 0)),
            pl.BlockSpec((8, 128), lambda i: (0, 0)),
        ],
        out_specs=[
            pl.BlockSpec((B, 128), lambda i: (i, 0)),
            pl.BlockSpec((8, 128), lambda i: (0, 0)),
        ],
        out_shape=[
            jax.ShapeDtypeStruct((npad, 128), F32),
            jax.ShapeDtypeStruct((8, 128), F32),
        ],
        compiler_params=pltpu.CompilerParams(dimension_semantics=("arbitrary",)),
    )(hin, agg_a, agg_b, W1, b1, W2f, b2f, ST)


def _tc_head(gs, gd, eap, ebp, ST2, M1, be1, We2, be2, We3, be3,
             Wh1a, Wh1b, Wh1c, bh1, Wh2, bh2, Wh3, bh3, wh4, misc,
             epad, e_count, ef, e_off, g):
    B = 512
    grid = epad // B

    def body(gs_ref, gd_ref, ea_ref, eb_ref, st_ref, m1_ref, be1_ref,
             we2_ref, be2_ref, we3_ref, be3_ref, wa_ref, wb_ref, wc_ref,
             bh1_ref, wh2_ref, bh2_ref, wh3_ref, bh3_ref, wh4_ref, misc_ref,
             out_ref):
        pid = pl.program_id(0)
        ea = ea_ref[...]
        iote = lax.broadcasted_iota(jnp.int32, (1, ef), 1)
        acc = jnp.zeros((B, ef), F32)
        for i in range(5):
            gi = ea[:, i:i + 1].astype(jnp.int32) + e_off[i]
            acc = acc + (gi == iote).astype(F32)
        acc = acc + ea[:, 5:6] * (iote == e_off[5]).astype(F32)
        e1 = _relu(jnp.dot(acc, m1_ref[...], preferred_element_type=F32)
                   + be1_ref[0:1, :])
        e2 = _relu(jnp.dot(e1, we2_ref[...], preferred_element_type=F32)
                   + be2_ref[0:1, :])
        e3 = jnp.dot(e2, we3_ref[...], preferred_element_type=F32) + be3_ref[0:1, :]
        S = st_ref[0:1, :]
        T = st_ref[1:2, :]
        xs = gs_ref[...] * S + T
        xd = gd_ref[...] * S + T
        z1 = _relu(jnp.dot(xs, wa_ref[...], preferred_element_type=F32)
                   + jnp.dot(xd, wb_ref[...], preferred_element_type=F32)
                   + jnp.dot(e3, wc_ref[...], preferred_element_type=F32)
                   + bh1_ref[0:1, :])
        z2 = _relu(jnp.dot(z1, wh2_ref[...], preferred_element_type=F32)
                   + bh2_ref[0:1, :])
        z3 = _relu(jnp.dot(z2, wh3_ref[...], preferred_element_type=F32)
                   + bh3_ref[0:1, :])
        z4 = jnp.sum(z3 * wh4_ref[0:1, :], axis=1, keepdims=True) \
            + misc_ref[0:1, 0:1]
        rid = lax.broadcasted_iota(jnp.int32, (B, 1), 0) + pid * B
        z4 = jnp.where(rid < e_count, z4, 0.0)
        iog = lax.broadcasted_iota(jnp.int32, (1, g), 1).astype(F32)
        onehot = (eb_ref[...] == iog).astype(F32)
        pooled = lax.dot_general(z4, onehot, (((0,), (0,)), ((), ())),
                                 preferred_element_type=F32)

        @pl.when(pid == 0)
        def _():
            out_ref[...] = jnp.zeros((8, g), F32)

        out_ref[...] = out_ref[...] + jnp.concatenate(
            [pooled, jnp.zeros((7, g), F32)], axis=0)

    full = lambda i: (0, 0)
    return pl.pallas_call(
        body,
        grid=(grid,),
        in_specs=[
            pl.BlockSpec((B, 128), lambda i: (i, 0)),
            pl.BlockSpec((B, 128), lambda i: (i, 0)),
            pl.BlockSpec((B, 6), lambda i: (i, 0)),
            pl.BlockSpec((B, 1), lambda i: (i, 0)),
            pl.BlockSpec((8, 128), full),
            pl.BlockSpec((ef, 22), full),
            pl.BlockSpec((8, 22), full),
            pl.BlockSpec((22, 40), full),
            pl.BlockSpec((8, 40), full),
            pl.BlockSpec((40, 40), full),
            pl.BlockSpec((8, 40), full),
            pl.BlockSpec((128, 512), full),
            pl.BlockSpec((128, 512), full),
            pl.BlockSpec((40, 512), full),
            pl.BlockSpec((8, 512), full),
            pl.BlockSpec((512, 512), full),
            pl.BlockSpec((8, 512), full),
            pl.BlockSpec((512, 256), full),
            pl.BlockSpec((8, 256), full),
            pl.BlockSpec((8, 256), full),
            pl.BlockSpec((8, 8), full),
        ],
        out_specs=pl.BlockSpec((8, g), full),
        out_shape=jax.ShapeDtypeStruct((8, g), F32),
        compiler_params=pltpu.CompilerParams(dimension_semantics=("arbitrary",)),
    )(gs, gd, eap, ebp, ST2, M1, be1, We2, be2, We3, be3,
      Wh1a, Wh1b, Wh1c, bh1, Wh2, bh2, Wh3, bh3, wh4, misc)


def _pad_bias(b, n):
    out = jnp.zeros((8, n), F32)
    return out.at[0, :b.shape[0]].set(b)


def kernel(x, edge_index, edge_attr, edge_batch, pos, params):
    n = x.shape[0]
    e = edge_index.shape[1]
    g = 64
    npad = -(-n // 1024) * 1024
    epad = -(-e // (NC * NS * CH)) * (NC * NS * CH)

    node_tabs = params['node_emb']
    edge_tabs = params['edge_emb']
    nv = [t.shape[0] for t in node_tabs]
    nd = [t.shape[1] for t in node_tabs]
    ev = [t.shape[0] for t in edge_tabs]
    ed = [t.shape[1] for t in edge_tabs]
    n_voff = [0]
    for v in nv:
        n_voff.append(n_voff[-1] + v)
    n_doff = [0]
    for dd in nd:
        n_doff.append(n_doff[-1] + dd)
    e_voff = [0]
    for v in ev:
        e_voff.append(e_voff[-1] + v)
    e_doff = [0]
    for dd in ed:
        e_doff.append(e_doff[-1] + dd)
    nf = -(-(n_voff[-1] + 4) // 8) * 8
    ef = -(-(e_voff[-1] + 1) // 8) * 8

    T0 = jnp.zeros((nf, 128), F32)
    for i in range(9):
        T0 = T0.at[n_voff[i]:n_voff[i] + nv[i],
                   n_doff[i]:n_doff[i] + nd[i]].set(node_tabs[i])
    T0 = T0.at[n_voff[-1], n_doff[-1]].set(1.0)
    for j in range(3):
        T0 = T0.at[n_voff[-1] + 1 + j, n_doff[-1] + 1 + j].set(0.1)
    node_onehot_off = n_voff[:9] + [n_voff[-1]]
    edge_onehot_off = e_voff[:5] + [e_voff[-1]]

    ea_map = jnp.zeros((ef, 11), F32)
    for i in range(5):
        ea_map = ea_map.at[e_voff[i]:e_voff[i] + ev[i],
                           e_doff[i]:e_doff[i] + ed[i]].set(edge_tabs[i])
    ea_map = ea_map.at[e_voff[-1], 10].set(0.1)
    M1 = ea_map @ params['e_lin'][0]['w'].T

    W1 = jnp.zeros((128, 108), F32).at[:37].set(params['x_nn1'][0]['w'].T)
    b1 = _pad_bias(params['x_nn1'][0]['b'], 108)
    W2f = jnp.zeros((108, 128), F32).at[:, :108].set(params['x_nn1'][1]['w'].T)
    b2f = _pad_bias(params['x_nn1'][1]['b'], 128).at[0, 108].set(1.0)
    W3 = jnp.zeros((128, 108), F32).at[:108].set(params['x_nn2'][0]['w'].T)
    b3 = _pad_bias(params['x_nn2'][0]['b'], 108)
    W4f = jnp.zeros((108, 128), F32).at[:, :108].set(params['x_nn2'][1]['w'].T)
    b4f = _pad_bias(params['x_nn2'][1]['b'], 128)

    h0w = params['head'][0]['w']
    Wh1a = jnp.zeros((128, 512), F32).at[:108].set(h0w[:, :108].T)
    Wh1b = jnp.zeros((128, 512), F32).at[:108].set(h0w[:, 108:216].T)
    Wh1c = h0w[:, 216:256].T
    bh1 = _pad_bias(params['head'][0]['b'], 512)
    Wh2 = params['head'][1]['w'].T
    bh2 = _pad_bias(params['head'][1]['b'], 512)
    Wh3 = params['head'][2]['w'].T
    bh3 = _pad_bias(params['head'][2]['b'], 256)
    wh4 = jnp.zeros((8, 256), F32).at[0].set(params['head'][3]['w'][0])
    misc = jnp.zeros((8, 8), F32).at[0, 0].set(params['head'][3]['b'][0])
    be1 = _pad_bias(params['e_lin'][0]['b'], 22)
    We2 = params['e_lin'][1]['w'].T
    be2 = _pad_bias(params['e_lin'][1]['b'], 40)
    We3 = params['e_lin'][2]['w'].T
    be3 = _pad_bias(params['e_lin'][2]['b'], 40)

    xp = jnp.zeros((npad, 13), F32).at[:n].set(x)
    posp = jnp.zeros((npad, 3), F32).at[:n].set(pos)
    srcp = jnp.full((epad,), n, jnp.int32).at[:e].set(edge_index[0])
    dstp = jnp.full((epad,), n, jnp.int32).at[:e].set(edge_index[1])
    src2 = srcp.reshape(-1, CW)
    dst2 = dstp.reshape(-1, CW)
    src2s = srcp.reshape(-1, CWS)
    dst2s = dstp.reshape(-1, CWS)

    eap = jnp.zeros((epad, 6), F32).at[:e].set(edge_attr)
    ebp = jnp.zeros((epad, 1), F32).at[:e, 0].set(edge_batch.astype(F32))

    h0 = _tc_node_features(xp, posp, T0, npad, nf, node_onehot_off)
    agg0 = _sc_scatter_add(h0, src2s, dst2s, npad, epad)
    dummy_st = jnp.zeros((8, 128), F32)
    u1, st1 = _tc_gin(h0, agg0[:npad], agg0[npad:], W1, b1, W2f, b2f,
                      dummy_st, npad, n, fold_bn=False)
    mean1 = st1[0, :108] / n
    var1 = st1[1, :108] / n - mean1 * mean1
    s1 = params['bn1']['g'] / jnp.sqrt(var1 + 1e-5)
    t1 = params['bn1']['b'] - mean1 * s1
    ST1 = jnp.zeros((8, 128), F32).at[0, :108].set(s1).at[1, :108].set(t1)

    aggu = _sc_scatter_add(u1, src2s, dst2s, npad, epad)
    u2, st2 = _tc_gin(u1, aggu[:npad], aggu[npad:], W3, b3, W4f, b4f,
                      ST1, npad, n, fold_bn=True)
    mean2 = st2[0, :108] / n
    var2 = st2[1, :108] / n - mean2 * mean2
    s2 = params['bn2']['g'] / jnp.sqrt(var2 + 1e-5)
    t2 = params['bn2']['b'] - mean2 * s2
    ST2 = jnp.zeros((8, 128), F32).at[0, :108].set(s2).at[1, :108].set(t2)

    gs, gd = _sc_gather2(u2, src2s, dst2s, epad)
    pooled = _tc_head(gs, gd, eap, ebp, ST2, M1, be1, We2, be2, We3, be3,
                      Wh1a, Wh1b, Wh1c, bh1, Wh2, bh2, Wh3, bh3, wh4, misc,
                      epad, e, ef, edge_onehot_off, g)
    return pooled[0, :].reshape(g, 1)
